# Initial kernel scaffold; baseline (speedup 1.0000x reference)
import jax
import jax.numpy as jnp
from jax.experimental import pallas as pl


def kernel(x, d_emb, data_table, pos_table):
    raise NotImplementedError("write your pallas kernel here")


# speedup vs baseline: 1.1425x; 1.1425x over previous
import functools
import math

import jax
import jax.numpy as jnp
from jax import lax
from jax.experimental import pallas as pl
from jax.experimental.pallas import tpu as pltpu
from jax.experimental.pallas import tpu_sc as plsc


def _make_sc_kernel(V, D, N):
    info = plsc.get_sparse_core_info()
    NC, NS, L = info.num_cores, info.num_subcores, info.num_lanes
    NW = NC * NS
    assert N % NW == 0 and D % L == 0
    n_per_w = N // NW
    T = 32
    assert n_per_w % T == 0
    n_chunks = n_per_w // T
    d_vecs = D // L

    mesh = plsc.VectorSubcoreMesh(core_axis_name="c", subcore_axis_name="s")

    @functools.partial(
        pl.kernel,
        mesh=mesh,
        out_type=jax.ShapeDtypeStruct((N, D), jnp.float32),
        scratch_types=[
            pltpu.VMEM((n_per_w,), jnp.int32),
            pltpu.VMEM((T, D), jnp.float32),
            pltpu.VMEM((T, D), jnp.float32),
            pltpu.SemaphoreType.DMA,
            pltpu.SemaphoreType.DMA,
        ],
    )
    def k(data_hbm, pos_hbm, idx_hbm, out_hbm, idx_v, buf_a, buf_b, sem_a, sem_b):
        wid = lax.axis_index("s") * NC + lax.axis_index("c")
        base = wid * n_per_w
        pltpu.sync_copy(idx_hbm.at[pl.ds(base, n_per_w)], idx_v)
        scale = math.sqrt(float(D))

        for c in range(n_chunks):
            idx_c = idx_v.at[pl.ds(c * T, T)]
            cp_a = pltpu.async_copy(data_hbm.at[idx_c], buf_a, sem_a)
            cp_b = pltpu.async_copy(pos_hbm.at[idx_c], buf_b, sem_b)
            cp_a.wait()
            cp_b.wait()

            def body(t, _):
                for j in range(d_vecs):
                    sl = pl.ds(j * L, L)
                    a = buf_a[t, sl]
                    b = buf_b[t, sl]
                    buf_a[t, sl] = (a + b) * scale
                return 0

            lax.fori_loop(0, T, body, 0)
            pltpu.sync_copy(buf_a, out_hbm.at[pl.ds(base + c * T, T)])

    return k


@functools.lru_cache(maxsize=None)
def _get_kernel(V, D, N):
    return _make_sc_kernel(V, D, N)


def kernel(x, d_emb, data_table, pos_table):
    B, S = x.shape
    V, D = data_table.shape
    idx = x.reshape(B * S).astype(jnp.int32)
    k = _get_kernel(V, D, B * S)
    out = k(data_table, pos_table, idx)
    return out.reshape(B, S, D)


# speedup vs baseline: 1.6076x; 1.4070x over previous
import functools
import math

import jax
import jax.numpy as jnp
from jax import lax
from jax.experimental import pallas as pl
from jax.experimental.pallas import tpu as pltpu
from jax.experimental.pallas import tpu_sc as plsc


def _make_sc_kernel(V, D, N):
    info = plsc.get_sparse_core_info()
    NC, NS, L = info.num_cores, info.num_subcores, info.num_lanes
    NW = NC * NS
    assert N % NW == 0 and D % L == 0
    n_per_w = N // NW
    T = 16
    NB = 3
    assert n_per_w % T == 0
    n_chunks = n_per_w // T
    d_vecs = D // L
    scale = math.sqrt(float(D))

    mesh = plsc.VectorSubcoreMesh(core_axis_name="c", subcore_axis_name="s")

    scratch = [pltpu.VMEM((n_per_w,), jnp.int32)]
    scratch += [pltpu.VMEM((T, D), jnp.float32) for _ in range(2 * NB)]
    scratch += [pltpu.SemaphoreType.DMA for _ in range(3 * NB)]

    @functools.partial(
        pl.kernel,
        mesh=mesh,
        out_type=jax.ShapeDtypeStruct((N, D), jnp.float32),
        scratch_types=scratch,
    )
    def k(data_hbm, pos_hbm, idx_hbm, out_hbm, idx_v, *bufs_and_sems):
        A = bufs_and_sems[0:NB]
        Bb = bufs_and_sems[NB:2 * NB]
        sga = bufs_and_sems[2 * NB:3 * NB]
        sgb = bufs_and_sems[3 * NB:4 * NB]
        ss = bufs_and_sems[4 * NB:5 * NB]

        wid = lax.axis_index("s") * NC + lax.axis_index("c")
        base = wid * n_per_w
        pltpu.sync_copy(idx_hbm.at[pl.ds(base, n_per_w)], idx_v)

        def start_g(c, b):
            idx_c = idx_v.at[pl.ds(c * T, T)]
            pltpu.async_copy(data_hbm.at[idx_c], A[b], sga[b])
            pltpu.async_copy(pos_hbm.at[idx_c], Bb[b], sgb[b])

        def wait_g(b):
            idx0 = idx_v.at[pl.ds(0, T)]
            pltpu.make_async_copy(data_hbm.at[idx0], A[b], sga[b]).wait()
            pltpu.make_async_copy(pos_hbm.at[idx0], Bb[b], sgb[b]).wait()

        def start_s(c, b):
            pltpu.async_copy(A[b], out_hbm.at[pl.ds(base + c * T, T)], ss[b])

        def wait_s(b):
            pltpu.make_async_copy(A[b], out_hbm.at[pl.ds(base, T)], ss[b]).wait()

        def compute(b):
            def body(t, _):
                for j in range(d_vecs):
                    sl = pl.ds(j * L, L)
                    A[b][t, sl] = (A[b][t, sl] + Bb[b][t, sl]) * scale
                return 0
            lax.fori_loop(0, T, body, 0)

        def phase(c, b, issue, store_wait):
            if issue:
                bw = (b + 2) % NB
                if store_wait:
                    wait_s(bw)
                start_g(c + 2, bw)
            wait_g(b)
            compute(b)
            start_s(c, b)

        start_g(0, 0)
        start_g(1, 1)
        phase(0, 0, issue=True, store_wait=False)
        phase(1, 1, issue=True, store_wait=True)

        n_mid = n_chunks - 4
        assert n_mid % NB == 0
        def mid(i, _):
            c2 = 2 + NB * i
            phase(c2, 2, issue=True, store_wait=True)
            phase(c2 + 1, 0, issue=True, store_wait=True)
            phase(c2 + 2, 1, issue=True, store_wait=True)
            return 0
        lax.fori_loop(0, n_mid // NB, mid, 0)

        phase(n_chunks - 2, (n_chunks - 2) % NB, issue=False, store_wait=False)
        phase(n_chunks - 1, (n_chunks - 1) % NB, issue=False, store_wait=False)
        for b in range(NB):
            wait_s(b)

    return k


@functools.lru_cache(maxsize=None)
def _get_kernel(V, D, N):
    return _make_sc_kernel(V, D, N)


def kernel(x, d_emb, data_table, pos_table):
    B, S = x.shape
    V, D = data_table.shape
    idx = x.reshape(B * S).astype(jnp.int32)
    k = _get_kernel(V, D, B * S)
    out = k(data_table, pos_table, idx)
    return out.reshape(B, S, D)


# speedup vs baseline: 1.6510x; 1.0270x over previous
import functools
import math

import jax
import jax.numpy as jnp
from jax import lax
from jax.experimental import pallas as pl
from jax.experimental.pallas import tpu as pltpu
from jax.experimental.pallas import tpu_sc as plsc


def _make_sc_kernel(V, D, N):
    info = plsc.get_sparse_core_info()
    NC, NS, L = info.num_cores, info.num_subcores, info.num_lanes
    NW = NC * NS
    assert N % NW == 0 and D % L == 0
    n_per_w = N // NW
    T = 16
    NB = 3
    assert n_per_w % T == 0
    n_chunks = n_per_w // T
    d_vecs = D // L
    scale = math.sqrt(float(D))
    n_groups = (n_chunks + NB - 1) // NB

    mesh = plsc.VectorSubcoreMesh(core_axis_name="c", subcore_axis_name="s")

    scratch = [pltpu.VMEM((n_per_w,), jnp.int32)]
    scratch += [pltpu.VMEM((T, D), jnp.float32) for _ in range(2 * NB)]
    scratch += [pltpu.SemaphoreType.DMA for _ in range(3 * NB)]

    @functools.partial(
        pl.kernel,
        mesh=mesh,
        out_type=jax.ShapeDtypeStruct((N, D), jnp.float32),
        scratch_types=scratch,
    )
    def k(data_hbm, pos_hbm, idx_hbm, out_hbm, idx_v, *bufs_and_sems):
        A = bufs_and_sems[0:NB]
        Bb = bufs_and_sems[NB:2 * NB]
        sga = bufs_and_sems[2 * NB:3 * NB]
        sgb = bufs_and_sems[3 * NB:4 * NB]
        ss = bufs_and_sems[4 * NB:5 * NB]

        wid = lax.axis_index("s") * NC + lax.axis_index("c")
        base = wid * n_per_w
        pltpu.sync_copy(idx_hbm.at[pl.ds(base, n_per_w)], idx_v)

        def start_g(c, b):
            idx_c = idx_v.at[pl.ds(c * T, T)]
            pltpu.async_copy(data_hbm.at[idx_c], A[b], sga[b])
            pltpu.async_copy(pos_hbm.at[idx_c], Bb[b], sgb[b])

        def wait_g(b):
            idx0 = idx_v.at[pl.ds(0, T)]
            pltpu.make_async_copy(data_hbm.at[idx0], A[b], sga[b]).wait()
            pltpu.make_async_copy(pos_hbm.at[idx0], Bb[b], sgb[b]).wait()

        def start_s(c, b):
            pltpu.async_copy(A[b], out_hbm.at[pl.ds(base + c * T, T)], ss[b])

        def wait_s(b):
            pltpu.make_async_copy(A[b], out_hbm.at[pl.ds(base, T)], ss[b]).wait()

        def compute(b):
            def body(t, _):
                for j in range(d_vecs):
                    sl = pl.ds(j * L, L)
                    A[b][t, sl] = (A[b][t, sl] + Bb[b][t, sl]) * scale
                return 0
            lax.fori_loop(0, T, body, 0)

        def phase(c, b):
            bw = (b + 2) % NB

            @pl.when(jnp.logical_and(c >= 1, c + 2 < n_chunks))
            def _():
                wait_s(bw)

            @pl.when(c + 2 < n_chunks)
            def _():
                start_g(c + 2, bw)

            @pl.when(c < n_chunks)
            def _():
                wait_g(b)
                compute(b)
                start_s(c, b)

        start_g(0, 0)
        start_g(1, 1)

        def grp(i, _):
            c0 = NB * i
            phase(c0, 0)
            phase(c0 + 1, 1)
            phase(c0 + 2, 2)
            return 0
        lax.fori_loop(0, n_groups, grp, 0)

        for b in range(NB):
            wait_s(b)

    return k


@functools.lru_cache(maxsize=None)
def _get_kernel(V, D, N):
    return _make_sc_kernel(V, D, N)


def kernel(x, d_emb, data_table, pos_table):
    B, S = x.shape
    V, D = data_table.shape
    idx = x.reshape(B * S).astype(jnp.int32)
    k = _get_kernel(V, D, B * S)
    out = k(data_table, pos_table, idx)
    return out.reshape(B, S, D)
